# Initial kernel scaffold; baseline (speedup 1.0000x reference)
import jax
import jax.numpy as jnp
from jax.experimental import pallas as pl


def kernel(x, edge_index, W1, b1, W2, b2):
    raise NotImplementedError("write your pallas kernel here")


# speedup vs baseline: 4.4842x; 4.4842x over previous
import functools

import jax
import jax.numpy as jnp
from jax import lax
from jax.experimental import pallas as pl
from jax.experimental.pallas import tpu as pltpu
from jax.experimental.pallas import tpu_sc as plsc

NC = 2
NS = 16
NW = NC * NS
LN = 16
CH = 128
NH = 2


def _ceil_to(a, m):
    return -(-a // m) * m


def _row_chunks(total, mx):
    out = []
    while total > 0:
        sz = min(mx, total)
        out.append(sz)
        total -= sz
    return out


def _deg_body(NCH, N_pad, src_hbm, dst_hbm, dego_hbm, degi_hbm,
              idx_s, idx_d, ones_v, zero_v, dego_sh, degi_sh):
    c = lax.axis_index("c")
    s = lax.axis_index("s")
    w = c * NS + s
    rpt = N_pad // NS
    base = s * rpt

    for j in range(CH // LN):
        ones_v[pl.ds(j * LN, LN)] = jnp.ones((LN,), jnp.float32)
        zero_v[pl.ds(j * LN, LN)] = jnp.zeros((LN,), jnp.float32)

    off = 0
    for sz in _row_chunks(rpt, CH):
        pltpu.sync_copy(zero_v.at[pl.ds(0, sz)], dego_sh.at[pl.ds(base + off, sz)])
        pltpu.sync_copy(zero_v.at[pl.ds(0, sz)], degi_sh.at[pl.ds(base + off, sz)])
        off += sz
    pltpu.sync_copy(src_hbm.at[w], idx_s)
    pltpu.sync_copy(dst_hbm.at[w], idx_d)
    plsc.subcore_barrier()

    def chunk(j, carry):
        pltpu.sync_copy(ones_v, dego_sh.at[idx_s.at[j]], add=True)
        pltpu.sync_copy(ones_v, degi_sh.at[idx_d.at[j]], add=True)
        return carry

    lax.fori_loop(0, NCH, chunk, 0)
    plsc.subcore_barrier()

    off = 0
    for sz in _row_chunks(rpt, CH):
        pltpu.sync_copy(dego_sh.at[pl.ds(base + off, sz)], ones_v.at[pl.ds(0, sz)])
        pltpu.sync_copy(ones_v.at[pl.ds(0, sz)],
                        dego_hbm.at[pl.ds(c * N_pad + base + off, sz)])
        pltpu.sync_copy(degi_sh.at[pl.ds(base + off, sz)], zero_v.at[pl.ds(0, sz)])
        pltpu.sync_copy(zero_v.at[pl.ds(0, sz)],
                        degi_hbm.at[pl.ds(c * N_pad + base + off, sz)])
        off += sz


def _mp_body(NCH, N_pad, DH, f0_hbm, f1_hbm, src_hbm, dst_hbm, out_hbm,
             idx_s, idx_d, b0, b1, sem0, sem1, agg_sh):
    c = lax.axis_index("c")
    s = lax.axis_index("s")
    w = c * NS + s
    rpt = N_pad // NS
    base = s * rpt
    feats = [f0_hbm, f1_hbm]

    pltpu.sync_copy(src_hbm.at[w], idx_s)
    pltpu.sync_copy(dst_hbm.at[w], idx_d)

    for h in range(NH):
        feat = feats[h]

        def zrow(i, carry):
            for j in range(DH // LN):
                b0[i, pl.ds(j * LN, LN)] = jnp.zeros((LN,), jnp.float32)
            return carry

        lax.fori_loop(0, CH, zrow, 0)
        off = 0
        for sz in _row_chunks(rpt, CH):
            pltpu.sync_copy(b0.at[pl.ds(0, sz)], agg_sh.at[pl.ds(base + off, sz)])
            off += sz
        plsc.subcore_barrier()

        pltpu.async_copy(feat.at[idx_s.at[0]], b0, sem0)

        def group(g, carry):
            j0 = g * 2
            j1 = j0 + 1

            @pl.when(j1 < NCH)
            def _():
                pltpu.async_copy(feat.at[idx_s.at[j1]], b1, sem1)

            pltpu.make_async_copy(feat.at[idx_s.at[j0]], b0, sem0).wait()
            pltpu.sync_copy(b0, agg_sh.at[idx_d.at[j0]], add=True)

            @pl.when(j1 < NCH)
            def _():
                @pl.when(j1 + 1 < NCH)
                def _():
                    pltpu.async_copy(feat.at[idx_s.at[j1 + 1]], b0, sem0)

                pltpu.make_async_copy(feat.at[idx_s.at[j1]], b1, sem1).wait()
                pltpu.sync_copy(b1, agg_sh.at[idx_d.at[j1]], add=True)

            return carry

        lax.fori_loop(0, (NCH + 1) // 2, group, 0)
        plsc.subcore_barrier()

        off = 0
        for sz in _row_chunks(rpt, CH):
            pltpu.sync_copy(agg_sh.at[pl.ds(base + off, sz)], b0.at[pl.ds(0, sz)])
            pltpu.sync_copy(b0.at[pl.ds(0, sz)],
                            out_hbm.at[h, c, pl.ds(base + off, sz)])
            off += sz
        plsc.subcore_barrier()


def _pre_body(DH, x_ref, do0, do1, xn0_ref, xn1_ref):
    n_out = lax.rsqrt(jnp.maximum(do0[...] + do1[...], 1.0))
    xn = x_ref[...] * n_out
    xn0_ref[...] = xn[:, :DH]
    xn1_ref[...] = xn[:, DH:]


def _layer1_body(n_valid, blk, DH, s00, s01, s10, s11, w_ref, b_ref,
                 di0, di1, do0, do1, zn0_ref, zn1_ref):
    i = pl.program_id(0)
    n_in = lax.rsqrt(jnp.maximum(di0[...] + di1[...], 1.0))
    z = (jnp.dot(s00[...] + s01[...], w_ref[:DH, :],
                 preferred_element_type=jnp.float32,
                 precision=lax.Precision.HIGHEST)
         + jnp.dot(s10[...] + s11[...], w_ref[DH:, :],
                   preferred_element_type=jnp.float32,
                   precision=lax.Precision.HIGHEST))
    z = jnp.maximum(z * n_in + b_ref[...], 0.0)
    n_out = lax.rsqrt(jnp.maximum(do0[...] + do1[...], 1.0))
    row = i * blk + lax.broadcasted_iota(jnp.int32, (blk, 1), 0)
    zn = jnp.where(row < n_valid, z * n_out, 0.0)
    zn0_ref[...] = zn[:, :DH]
    zn1_ref[...] = zn[:, DH:]


def _layer2_body(DH, s00, s01, s10, s11, w_ref, b_ref, di0, di1, out_ref):
    n_in = lax.rsqrt(jnp.maximum(di0[...] + di1[...], 1.0))
    z = (jnp.dot(s00[...] + s01[...], w_ref[:DH, :],
                 preferred_element_type=jnp.float32,
                 precision=lax.Precision.HIGHEST)
         + jnp.dot(s10[...] + s11[...], w_ref[DH:, :],
                   preferred_element_type=jnp.float32,
                   precision=lax.Precision.HIGHEST))
    out_ref[...] = z * n_in + b_ref[...]


@jax.jit
def kernel(x, edge_index, W1, b1, W2, b2):
    N, D = x.shape
    E = edge_index.shape[1]
    DH = D // NH
    N_pad = _ceil_to(N + 1, CH)
    ept = _ceil_to(-(-E // NW), CH)
    NCH = ept // CH
    E_pad = ept * NW

    pad = jnp.full((E_pad - E,), N, jnp.int32)
    src = jnp.concatenate([edge_index[0], pad]).reshape(NW, NCH, CH)
    dst = jnp.concatenate([edge_index[1], pad]).reshape(NW, NCH, CH)
    x_pad = jnp.pad(x, ((0, N_pad - N), (0, 0)))

    mesh = plsc.VectorSubcoreMesh(core_axis_name="c", subcore_axis_name="s",
                                  num_cores=NC, num_subcores=NS)

    deg_call = pl.kernel(
        functools.partial(_deg_body, NCH, N_pad),
        out_type=[jax.ShapeDtypeStruct((NC * N_pad,), jnp.float32),
                  jax.ShapeDtypeStruct((NC * N_pad,), jnp.float32)],
        mesh=mesh,
        scratch_types=[
            pltpu.VMEM((NCH, CH), jnp.int32),
            pltpu.VMEM((NCH, CH), jnp.int32),
            pltpu.VMEM((CH,), jnp.float32),
            pltpu.VMEM((CH,), jnp.float32),
            pltpu.VMEM_SHARED((N_pad,), jnp.float32),
            pltpu.VMEM_SHARED((N_pad,), jnp.float32),
        ],
    )
    dego, degi = deg_call(src, dst)
    do0 = dego[:N_pad].reshape(N_pad, 1)
    do1 = dego[N_pad:].reshape(N_pad, 1)
    di0 = degi[:N_pad].reshape(N_pad, 1)
    di1 = degi[N_pad:].reshape(N_pad, 1)

    mp_call = pl.kernel(
        functools.partial(_mp_body, NCH, N_pad, DH),
        out_type=jax.ShapeDtypeStruct((NH, NC, N_pad, DH), jnp.float32),
        mesh=mesh,
        scratch_types=[
            pltpu.VMEM((NCH, CH), jnp.int32),
            pltpu.VMEM((NCH, CH), jnp.int32),
            pltpu.VMEM((CH, DH), jnp.float32),
            pltpu.VMEM((CH, DH), jnp.float32),
            pltpu.SemaphoreType.DMA,
            pltpu.SemaphoreType.DMA,
            pltpu.VMEM_SHARED((N_pad, DH), jnp.float32),
        ],
        compiler_params=pltpu.CompilerParams(use_tc_tiling_on_sc=False),
    )

    blk = CH
    grid = N_pad // blk
    col_spec = pl.BlockSpec((blk, 1), lambda i: (i, 0))
    mat_spec = pl.BlockSpec((blk, D), lambda i: (i, 0))
    half_spec = pl.BlockSpec((blk, DH), lambda i: (i, 0))
    w_spec = pl.BlockSpec((D, D), lambda i: (0, 0))
    b_spec = pl.BlockSpec((1, D), lambda i: (0, 0))

    xn0, xn1 = pl.pallas_call(
        functools.partial(_pre_body, DH),
        grid=(grid,),
        in_specs=[mat_spec, col_spec, col_spec],
        out_specs=[half_spec, half_spec],
        out_shape=[jax.ShapeDtypeStruct((N_pad, DH), jnp.float32),
                   jax.ShapeDtypeStruct((N_pad, DH), jnp.float32)],
    )(x_pad, do0, do1)

    S1 = mp_call(xn0, xn1, src, dst)

    zn0, zn1 = pl.pallas_call(
        functools.partial(_layer1_body, N, blk, DH),
        grid=(grid,),
        in_specs=[half_spec, half_spec, half_spec, half_spec, w_spec, b_spec,
                  col_spec, col_spec, col_spec, col_spec],
        out_specs=[half_spec, half_spec],
        out_shape=[jax.ShapeDtypeStruct((N_pad, DH), jnp.float32),
                   jax.ShapeDtypeStruct((N_pad, DH), jnp.float32)],
    )(S1[0, 0], S1[0, 1], S1[1, 0], S1[1, 1], W1, b1.reshape(1, D),
      di0, di1, do0, do1)

    S2 = mp_call(zn0, zn1, src, dst)

    out = pl.pallas_call(
        functools.partial(_layer2_body, DH),
        grid=(grid,),
        in_specs=[half_spec, half_spec, half_spec, half_spec, w_spec, b_spec,
                  col_spec, col_spec],
        out_specs=mat_spec,
        out_shape=jax.ShapeDtypeStruct((N_pad, D), jnp.float32),
    )(S2[0, 0], S2[0, 1], S2[1, 0], S2[1, 1], W2, b2.reshape(1, D), di0, di1)

    return out[:N]


# speedup vs baseline: 5.5048x; 1.2276x over previous
import functools

import jax
import jax.numpy as jnp
from jax import lax
from jax.experimental import pallas as pl
from jax.experimental.pallas import tpu as pltpu
from jax.experimental.pallas import tpu_sc as plsc

NC = 2
NS = 16
LN = 16
CH = 128
NH = 2
C0 = 102
C1 = 56


def _ceil_to(a, m):
    return -(-a // m) * m


def _row_chunks(total, mx):
    out = []
    while total > 0:
        sz = min(mx, total)
        out.append(sz)
        total -= sz
    return out


def _deg_body(N_pad, src_hbm, dst_hbm, dego_hbm, degi_hbm,
              idx_s, idx_d, ones_v, zero_v, dego_sh, degi_sh):
    c = lax.axis_index("c")
    s = lax.axis_index("s")
    rpt = N_pad // NS
    base = s * rpt

    for j in range(CH // LN):
        ones_v[pl.ds(j * LN, LN)] = jnp.ones((LN,), jnp.float32)
        zero_v[pl.ds(j * LN, LN)] = jnp.zeros((LN,), jnp.float32)

    off = 0
    for sz in _row_chunks(rpt, CH):
        pltpu.sync_copy(zero_v.at[pl.ds(0, sz)], dego_sh.at[pl.ds(base + off, sz)])
        pltpu.sync_copy(zero_v.at[pl.ds(0, sz)], degi_sh.at[pl.ds(base + off, sz)])
        off += sz

    def hist(nch, row_base):
        pltpu.sync_copy(src_hbm.at[pl.ds(row_base, nch)], idx_s.at[pl.ds(0, nch)])
        pltpu.sync_copy(dst_hbm.at[pl.ds(row_base, nch)], idx_d.at[pl.ds(0, nch)])
        plsc.subcore_barrier()

        def chunk(j, carry):
            pltpu.sync_copy(ones_v, dego_sh.at[idx_s.at[j]], add=True)
            pltpu.sync_copy(ones_v, degi_sh.at[idx_d.at[j]], add=True)
            return carry

        lax.fori_loop(0, nch, chunk, 0)

    @pl.when(c == 0)
    def _():
        hist(C0, s * C0)

    @pl.when(c == 1)
    def _():
        hist(C1, NS * C0 + s * C1)

    plsc.subcore_barrier()

    off = 0
    for sz in _row_chunks(rpt, CH):
        pltpu.sync_copy(dego_sh.at[pl.ds(base + off, sz)], ones_v.at[pl.ds(0, sz)])
        pltpu.sync_copy(ones_v.at[pl.ds(0, sz)],
                        dego_hbm.at[pl.ds(c * N_pad + base + off, sz)])
        pltpu.sync_copy(degi_sh.at[pl.ds(base + off, sz)], zero_v.at[pl.ds(0, sz)])
        pltpu.sync_copy(zero_v.at[pl.ds(0, sz)],
                        degi_hbm.at[pl.ds(c * N_pad + base + off, sz)])
        off += sz


def _mp_body(N_pad, DH, f0_hbm, f1_hbm, src_hbm, dst_hbm, out_hbm,
             idx_s, idx_d, b0, b1, sem0, sem1, agg_sh):
    c = lax.axis_index("c")
    s = lax.axis_index("s")
    rpt = N_pad // NS
    base = s * rpt
    feats = [f0_hbm, f1_hbm]

    def pipeline(feat, nch, row_base):
        pltpu.sync_copy(src_hbm.at[pl.ds(row_base, nch)], idx_s.at[pl.ds(0, nch)])
        pltpu.sync_copy(dst_hbm.at[pl.ds(row_base, nch)], idx_d.at[pl.ds(0, nch)])
        pltpu.async_copy(feat.at[idx_s.at[0]], b0, sem0)

        def group(g, carry):
            j0 = g * 2
            j1 = j0 + 1

            @pl.when(j1 < nch)
            def _():
                pltpu.async_copy(feat.at[idx_s.at[j1]], b1, sem1)

            pltpu.make_async_copy(feat.at[idx_s.at[j0]], b0, sem0).wait()
            pltpu.sync_copy(b0, agg_sh.at[idx_d.at[j0]], add=True)

            @pl.when(j1 < nch)
            def _():
                @pl.when(j1 + 1 < nch)
                def _():
                    pltpu.async_copy(feat.at[idx_s.at[j1 + 1]], b0, sem0)

                pltpu.make_async_copy(feat.at[idx_s.at[j1]], b1, sem1).wait()
                pltpu.sync_copy(b1, agg_sh.at[idx_d.at[j1]], add=True)

            return carry

        lax.fori_loop(0, (nch + 1) // 2, group, 0)

    for h in range(NH):
        def zrow(i, carry):
            for j in range(DH // LN):
                b0[i, pl.ds(j * LN, LN)] = jnp.zeros((LN,), jnp.float32)
            return carry

        lax.fori_loop(0, CH, zrow, 0)
        off = 0
        for sz in _row_chunks(rpt, CH):
            pltpu.sync_copy(b0.at[pl.ds(0, sz)], agg_sh.at[pl.ds(base + off, sz)])
            off += sz
        plsc.subcore_barrier()

        @pl.when(c == 0)
        def _():
            pipeline(feats[h], C0, s * C0)

        @pl.when(c == 1)
        def _():
            pipeline(feats[h], C1, NS * C0 + s * C1)

        plsc.subcore_barrier()

        off = 0
        for sz in _row_chunks(rpt, CH):
            pltpu.sync_copy(agg_sh.at[pl.ds(base + off, sz)], b0.at[pl.ds(0, sz)])
            pltpu.sync_copy(b0.at[pl.ds(0, sz)],
                            out_hbm.at[h, c, pl.ds(base + off, sz)])
            off += sz
        plsc.subcore_barrier()


def _pre_body(DH, x_ref, do0, do1, xn0_ref, xn1_ref):
    n_out = lax.rsqrt(jnp.maximum(do0[...] + do1[...], 1.0))
    xn = x_ref[...] * n_out
    xn0_ref[...] = xn[:, :DH]
    xn1_ref[...] = xn[:, DH:]


def _layer1_body(n_valid, DH, s00, s01, s10, s11, w_ref, b_ref,
                 di0, di1, do0, do1, zn0_ref, zn1_ref):
    n_in = lax.rsqrt(jnp.maximum(di0[...] + di1[...], 1.0))
    z = (jnp.dot(s00[...] + s01[...], w_ref[:DH, :],
                 preferred_element_type=jnp.float32,
                 precision=lax.Precision.HIGHEST)
         + jnp.dot(s10[...] + s11[...], w_ref[DH:, :],
                   preferred_element_type=jnp.float32,
                   precision=lax.Precision.HIGHEST))
    z = jnp.maximum(z * n_in + b_ref[...], 0.0)
    n_out = lax.rsqrt(jnp.maximum(do0[...] + do1[...], 1.0))
    blk = z.shape[0]
    row = pl.program_id(0) * blk + lax.broadcasted_iota(jnp.int32, (blk, 1), 0)
    zn = jnp.where(row < n_valid, z * n_out, 0.0)
    zn0_ref[...] = zn[:, :DH]
    zn1_ref[...] = zn[:, DH:]


def _layer2_body(DH, s00, s01, s10, s11, w_ref, b_ref, di0, di1, out_ref):
    n_in = lax.rsqrt(jnp.maximum(di0[...] + di1[...], 1.0))
    z = (jnp.dot(s00[...] + s01[...], w_ref[:DH, :],
                 preferred_element_type=jnp.float32,
                 precision=lax.Precision.HIGHEST)
         + jnp.dot(s10[...] + s11[...], w_ref[DH:, :],
                   preferred_element_type=jnp.float32,
                   precision=lax.Precision.HIGHEST))
    out_ref[...] = z * n_in + b_ref[...]


@jax.jit
def kernel(x, edge_index, W1, b1, W2, b2):
    N, D = x.shape
    E = edge_index.shape[1]
    DH = D // NH
    N_pad = _ceil_to(N + 1, CH)
    CMAX = max(C0, C1)
    E0 = NS * C0 * CH
    E_pad = NS * (C0 + C1) * CH
    assert E0 < E <= E_pad

    pad = jnp.full((E_pad - E,), N, jnp.int32)
    src = jnp.concatenate([edge_index[0], pad]).reshape(-1, CH)
    dst = jnp.concatenate([edge_index[1], pad]).reshape(-1, CH)
    x_pad = jnp.pad(x, ((0, N_pad - N), (0, 0)))

    mesh = plsc.VectorSubcoreMesh(core_axis_name="c", subcore_axis_name="s",
                                  num_cores=NC, num_subcores=NS)

    deg_call = pl.kernel(
        functools.partial(_deg_body, N_pad),
        out_type=[jax.ShapeDtypeStruct((NC * N_pad,), jnp.float32),
                  jax.ShapeDtypeStruct((NC * N_pad,), jnp.float32)],
        mesh=mesh,
        scratch_types=[
            pltpu.VMEM((CMAX, CH), jnp.int32),
            pltpu.VMEM((CMAX, CH), jnp.int32),
            pltpu.VMEM((CH,), jnp.float32),
            pltpu.VMEM((CH,), jnp.float32),
            pltpu.VMEM_SHARED((N_pad,), jnp.float32),
            pltpu.VMEM_SHARED((N_pad,), jnp.float32),
        ],
        compiler_params=pltpu.CompilerParams(use_tc_tiling_on_sc=False),
    )
    dego, degi = deg_call(src, dst)
    do0 = dego[:N_pad].reshape(N_pad, 1)
    do1 = dego[N_pad:].reshape(N_pad, 1)
    di0 = degi[:N_pad].reshape(N_pad, 1)
    di1 = degi[N_pad:].reshape(N_pad, 1)

    mp_call = pl.kernel(
        functools.partial(_mp_body, N_pad, DH),
        out_type=jax.ShapeDtypeStruct((NH, NC, N_pad, DH), jnp.float32),
        mesh=mesh,
        scratch_types=[
            pltpu.VMEM((CMAX, CH), jnp.int32),
            pltpu.VMEM((CMAX, CH), jnp.int32),
            pltpu.VMEM((CH, DH), jnp.float32),
            pltpu.VMEM((CH, DH), jnp.float32),
            pltpu.SemaphoreType.DMA,
            pltpu.SemaphoreType.DMA,
            pltpu.VMEM_SHARED((N_pad, DH), jnp.float32),
        ],
        compiler_params=pltpu.CompilerParams(use_tc_tiling_on_sc=False),
    )

    tc_grid = 8
    blk = N_pad // tc_grid
    col_spec = pl.BlockSpec((blk, 1), lambda i: (i, 0))
    mat_spec = pl.BlockSpec((blk, D), lambda i: (i, 0))
    half_spec = pl.BlockSpec((blk, DH), lambda i: (i, 0))
    w_spec = pl.BlockSpec((D, D), lambda i: (0, 0))
    b_spec = pl.BlockSpec((1, D), lambda i: (0, 0))

    xn0, xn1 = pl.pallas_call(
        functools.partial(_pre_body, DH),
        grid=(tc_grid,),
        in_specs=[mat_spec, col_spec, col_spec],
        out_specs=[half_spec, half_spec],
        out_shape=[jax.ShapeDtypeStruct((N_pad, DH), jnp.float32),
                   jax.ShapeDtypeStruct((N_pad, DH), jnp.float32)],
    )(x_pad, do0, do1)

    S1 = mp_call(xn0, xn1, src, dst)

    zn0, zn1 = pl.pallas_call(
        functools.partial(_layer1_body, N, DH),
        grid=(tc_grid,),
        in_specs=[half_spec, half_spec, half_spec, half_spec, w_spec, b_spec,
                  col_spec, col_spec, col_spec, col_spec],
        out_specs=[half_spec, half_spec],
        out_shape=[jax.ShapeDtypeStruct((N_pad, DH), jnp.float32),
                   jax.ShapeDtypeStruct((N_pad, DH), jnp.float32)],
    )(S1[0, 0], S1[0, 1], S1[1, 0], S1[1, 1], W1, b1.reshape(1, D),
      di0, di1, do0, do1)

    S2 = mp_call(zn0, zn1, src, dst)

    out = pl.pallas_call(
        functools.partial(_layer2_body, DH),
        grid=(tc_grid,),
        in_specs=[half_spec, half_spec, half_spec, half_spec, w_spec, b_spec,
                  col_spec, col_spec],
        out_specs=mat_spec,
        out_shape=jax.ShapeDtypeStruct((N_pad, D), jnp.float32),
    )(S2[0, 0], S2[0, 1], S2[1, 0], S2[1, 1], W2, b2.reshape(1, D), di0, di1)

    return out[:N]


# speedup vs baseline: 5.6776x; 1.0314x over previous
import functools

import jax
import jax.numpy as jnp
from jax import lax
from jax.experimental import pallas as pl
from jax.experimental.pallas import tpu as pltpu
from jax.experimental.pallas import tpu_sc as plsc

NC = 2
NS = 16
LN = 16
CH = 128
NH = 2
C0 = 116
C1 = 42


def _ceil_to(a, m):
    return -(-a // m) * m


def _row_chunks(total, mx):
    out = []
    while total > 0:
        sz = min(mx, total)
        out.append(sz)
        total -= sz
    return out


def _deg_body(N_pad, src_hbm, dst_hbm, dego_hbm, degi_hbm,
              idx_s, idx_d, ones_v, zero_v, dego_sh, degi_sh):
    c = lax.axis_index("c")
    s = lax.axis_index("s")
    rpt = N_pad // NS
    base = s * rpt

    for j in range(CH // LN):
        ones_v[pl.ds(j * LN, LN)] = jnp.ones((LN,), jnp.float32)
        zero_v[pl.ds(j * LN, LN)] = jnp.zeros((LN,), jnp.float32)

    off = 0
    for sz in _row_chunks(rpt, CH):
        pltpu.sync_copy(zero_v.at[pl.ds(0, sz)], dego_sh.at[pl.ds(base + off, sz)])
        pltpu.sync_copy(zero_v.at[pl.ds(0, sz)], degi_sh.at[pl.ds(base + off, sz)])
        off += sz

    def hist(nch, row_base):
        pltpu.sync_copy(src_hbm.at[pl.ds(row_base, nch)], idx_s.at[pl.ds(0, nch)])
        pltpu.sync_copy(dst_hbm.at[pl.ds(row_base, nch)], idx_d.at[pl.ds(0, nch)])
        plsc.subcore_barrier()

        def chunk(j, carry):
            pltpu.sync_copy(ones_v, dego_sh.at[idx_s.at[j]], add=True)
            pltpu.sync_copy(ones_v, degi_sh.at[idx_d.at[j]], add=True)
            return carry

        lax.fori_loop(0, nch, chunk, 0)

    @pl.when(c == 0)
    def _():
        hist(C0, s * C0)

    @pl.when(c == 1)
    def _():
        hist(C1, NS * C0 + s * C1)

    plsc.subcore_barrier()

    off = 0
    for sz in _row_chunks(rpt, CH):
        pltpu.sync_copy(dego_sh.at[pl.ds(base + off, sz)], ones_v.at[pl.ds(0, sz)])
        pltpu.sync_copy(ones_v.at[pl.ds(0, sz)],
                        dego_hbm.at[pl.ds(c * N_pad + base + off, sz)])
        pltpu.sync_copy(degi_sh.at[pl.ds(base + off, sz)], zero_v.at[pl.ds(0, sz)])
        pltpu.sync_copy(zero_v.at[pl.ds(0, sz)],
                        degi_hbm.at[pl.ds(c * N_pad + base + off, sz)])
        off += sz


def _mp_body(N_pad, DH, f0_hbm, f1_hbm, src_hbm, dst_hbm, out_hbm,
             idx_s, idx_d, b0, b1, sem0, sem1, agg_sh):
    c = lax.axis_index("c")
    s = lax.axis_index("s")
    rpt = N_pad // NS
    base = s * rpt
    feats = [f0_hbm, f1_hbm]

    def pipeline(feat, nch, row_base):
        pltpu.sync_copy(src_hbm.at[pl.ds(row_base, nch)], idx_s.at[pl.ds(0, nch)])
        pltpu.sync_copy(dst_hbm.at[pl.ds(row_base, nch)], idx_d.at[pl.ds(0, nch)])
        pltpu.async_copy(feat.at[idx_s.at[0]], b0, sem0)

        def group(g, carry):
            j0 = g * 2
            j1 = j0 + 1

            @pl.when(j1 < nch)
            def _():
                pltpu.async_copy(feat.at[idx_s.at[j1]], b1, sem1)

            pltpu.make_async_copy(feat.at[idx_s.at[j0]], b0, sem0).wait()
            pltpu.sync_copy(b0, agg_sh.at[idx_d.at[j0]], add=True)

            @pl.when(j1 < nch)
            def _():
                @pl.when(j1 + 1 < nch)
                def _():
                    pltpu.async_copy(feat.at[idx_s.at[j1 + 1]], b0, sem0)

                pltpu.make_async_copy(feat.at[idx_s.at[j1]], b1, sem1).wait()
                pltpu.sync_copy(b1, agg_sh.at[idx_d.at[j1]], add=True)

            return carry

        lax.fori_loop(0, (nch + 1) // 2, group, 0)

    for h in range(NH):
        def zrow(i, carry):
            for j in range(DH // LN):
                b0[i, pl.ds(j * LN, LN)] = jnp.zeros((LN,), jnp.float32)
            return carry

        lax.fori_loop(0, CH, zrow, 0)
        off = 0
        for sz in _row_chunks(rpt, CH):
            pltpu.sync_copy(b0.at[pl.ds(0, sz)], agg_sh.at[pl.ds(base + off, sz)])
            off += sz
        plsc.subcore_barrier()

        @pl.when(c == 0)
        def _():
            pipeline(feats[h], C0, s * C0)

        @pl.when(c == 1)
        def _():
            pipeline(feats[h], C1, NS * C0 + s * C1)

        plsc.subcore_barrier()

        off = 0
        for sz in _row_chunks(rpt, CH):
            pltpu.sync_copy(agg_sh.at[pl.ds(base + off, sz)], b0.at[pl.ds(0, sz)])
            pltpu.sync_copy(b0.at[pl.ds(0, sz)],
                            out_hbm.at[h, c, pl.ds(base + off, sz)])
            off += sz
        plsc.subcore_barrier()


def _pre_body(DH, x_ref, do0, do1, xn0_ref, xn1_ref):
    n_out = lax.rsqrt(jnp.maximum(do0[...] + do1[...], 1.0))
    xn = x_ref[...] * n_out
    xn0_ref[...] = xn[:, :DH]
    xn1_ref[...] = xn[:, DH:]


def _layer1_body(n_valid, DH, s00, s01, s10, s11, w_ref, b_ref,
                 di0, di1, do0, do1, zn0_ref, zn1_ref):
    n_in = lax.rsqrt(jnp.maximum(di0[...] + di1[...], 1.0))
    z = (jnp.dot(s00[...] + s01[...], w_ref[:DH, :],
                 preferred_element_type=jnp.float32,
                 precision=lax.Precision.HIGHEST)
         + jnp.dot(s10[...] + s11[...], w_ref[DH:, :],
                   preferred_element_type=jnp.float32,
                   precision=lax.Precision.HIGHEST))
    z = jnp.maximum(z * n_in + b_ref[...], 0.0)
    n_out = lax.rsqrt(jnp.maximum(do0[...] + do1[...], 1.0))
    blk = z.shape[0]
    row = pl.program_id(0) * blk + lax.broadcasted_iota(jnp.int32, (blk, 1), 0)
    zn = jnp.where(row < n_valid, z * n_out, 0.0)
    zn0_ref[...] = zn[:, :DH]
    zn1_ref[...] = zn[:, DH:]


def _layer2_body(DH, s00, s01, s10, s11, w_ref, b_ref, di0, di1, out_ref):
    n_in = lax.rsqrt(jnp.maximum(di0[...] + di1[...], 1.0))
    z = (jnp.dot(s00[...] + s01[...], w_ref[:DH, :],
                 preferred_element_type=jnp.float32,
                 precision=lax.Precision.HIGHEST)
         + jnp.dot(s10[...] + s11[...], w_ref[DH:, :],
                   preferred_element_type=jnp.float32,
                   precision=lax.Precision.HIGHEST))
    out_ref[...] = z * n_in + b_ref[...]


@jax.jit
def kernel(x, edge_index, W1, b1, W2, b2):
    N, D = x.shape
    E = edge_index.shape[1]
    DH = D // NH
    N_pad = _ceil_to(N + 1, CH)
    CMAX = max(C0, C1)
    E0 = NS * C0 * CH
    E_pad = NS * (C0 + C1) * CH
    assert E0 < E <= E_pad

    pad = jnp.full((E_pad - E,), N, jnp.int32)
    src = jnp.concatenate([edge_index[0], pad]).reshape(-1, CH)
    dst = jnp.concatenate([edge_index[1], pad]).reshape(-1, CH)
    x_pad = jnp.pad(x, ((0, N_pad - N), (0, 0)))

    mesh = plsc.VectorSubcoreMesh(core_axis_name="c", subcore_axis_name="s",
                                  num_cores=NC, num_subcores=NS)

    deg_call = pl.kernel(
        functools.partial(_deg_body, N_pad),
        out_type=[jax.ShapeDtypeStruct((NC * N_pad,), jnp.float32),
                  jax.ShapeDtypeStruct((NC * N_pad,), jnp.float32)],
        mesh=mesh,
        scratch_types=[
            pltpu.VMEM((CMAX, CH), jnp.int32),
            pltpu.VMEM((CMAX, CH), jnp.int32),
            pltpu.VMEM((CH,), jnp.float32),
            pltpu.VMEM((CH,), jnp.float32),
            pltpu.VMEM_SHARED((N_pad,), jnp.float32),
            pltpu.VMEM_SHARED((N_pad,), jnp.float32),
        ],
        compiler_params=pltpu.CompilerParams(use_tc_tiling_on_sc=False),
    )
    dego, degi = deg_call(src, dst)
    do0 = dego[:N_pad].reshape(N_pad, 1)
    do1 = dego[N_pad:].reshape(N_pad, 1)
    di0 = degi[:N_pad].reshape(N_pad, 1)
    di1 = degi[N_pad:].reshape(N_pad, 1)

    mp_call = pl.kernel(
        functools.partial(_mp_body, N_pad, DH),
        out_type=jax.ShapeDtypeStruct((NH, NC, N_pad, DH), jnp.float32),
        mesh=mesh,
        scratch_types=[
            pltpu.VMEM((CMAX, CH), jnp.int32),
            pltpu.VMEM((CMAX, CH), jnp.int32),
            pltpu.VMEM((CH, DH), jnp.float32),
            pltpu.VMEM((CH, DH), jnp.float32),
            pltpu.SemaphoreType.DMA,
            pltpu.SemaphoreType.DMA,
            pltpu.VMEM_SHARED((N_pad, DH), jnp.float32),
        ],
        compiler_params=pltpu.CompilerParams(use_tc_tiling_on_sc=False),
    )

    tc_grid = 8
    blk = N_pad // tc_grid
    col_spec = pl.BlockSpec((blk, 1), lambda i: (i, 0))
    mat_spec = pl.BlockSpec((blk, D), lambda i: (i, 0))
    half_spec = pl.BlockSpec((blk, DH), lambda i: (i, 0))
    w_spec = pl.BlockSpec((D, D), lambda i: (0, 0))
    b_spec = pl.BlockSpec((1, D), lambda i: (0, 0))

    xn0, xn1 = pl.pallas_call(
        functools.partial(_pre_body, DH),
        grid=(tc_grid,),
        in_specs=[mat_spec, col_spec, col_spec],
        out_specs=[half_spec, half_spec],
        out_shape=[jax.ShapeDtypeStruct((N_pad, DH), jnp.float32),
                   jax.ShapeDtypeStruct((N_pad, DH), jnp.float32)],
    )(x_pad, do0, do1)

    S1 = mp_call(xn0, xn1, src, dst)

    zn0, zn1 = pl.pallas_call(
        functools.partial(_layer1_body, N, DH),
        grid=(tc_grid,),
        in_specs=[half_spec, half_spec, half_spec, half_spec, w_spec, b_spec,
                  col_spec, col_spec, col_spec, col_spec],
        out_specs=[half_spec, half_spec],
        out_shape=[jax.ShapeDtypeStruct((N_pad, DH), jnp.float32),
                   jax.ShapeDtypeStruct((N_pad, DH), jnp.float32)],
    )(S1[0, 0], S1[0, 1], S1[1, 0], S1[1, 1], W1, b1.reshape(1, D),
      di0, di1, do0, do1)

    S2 = mp_call(zn0, zn1, src, dst)

    out = pl.pallas_call(
        functools.partial(_layer2_body, DH),
        grid=(tc_grid,),
        in_specs=[half_spec, half_spec, half_spec, half_spec, w_spec, b_spec,
                  col_spec, col_spec],
        out_specs=mat_spec,
        out_shape=jax.ShapeDtypeStruct((N_pad, D), jnp.float32),
    )(S2[0, 0], S2[0, 1], S2[1, 0], S2[1, 1], W2, b2.reshape(1, D), di0, di1)

    return out[:N]
